# Initial kernel scaffold; baseline (speedup 1.0000x reference)
import jax
import jax.numpy as jnp
from jax.experimental import pallas as pl


def kernel(x, efeat, edge_index, graph_ids, params):
    raise NotImplementedError("write your pallas kernel here")


# speedup vs baseline: 7.1490x; 7.1490x over previous
import functools

import jax
import jax.numpy as jnp
from jax import lax
from jax.experimental import pallas as pl
from jax.experimental.pallas import tpu as pltpu
from jax.experimental.pallas import tpu_sc as plsc

N = 10000
NP = 10240
E = 320000
DIN = 128
H = 64
DIM = 128
G = 512
NC = 2
NS = 16
NW = NC * NS
ECH = 128
NCHUNK = 79
EW = ECH * NCHUNK
EP = EW * NW
RT = NP // NS
NODE_W = NP // NW
NGC = NODE_W // 16
GS = G // NS
BLK = 1024

_mesh = plsc.VectorSubcoreMesh(
    core_axis_name="c", subcore_axis_name="s", num_cores=NC, num_subcores=NS)
_sc_params = pltpu.CompilerParams(
    use_tc_tiling_on_sc=False, needs_layout_passes=False)


def _f32(*shape):
  return jax.ShapeDtypeStruct(shape, jnp.float32)


@functools.partial(
    pl.kernel,
    out_type=_f32(NC, 2, NP, 16),
    mesh=_mesh,
    compiler_params=_sc_params,
    scratch_types=[
        pltpu.VMEM_SHARED((NP, 16), jnp.float32),
        pltpu.VMEM_SHARED((NP, 16), jnp.float32),
        pltpu.VMEM((NCHUNK, ECH), jnp.int32),
        pltpu.VMEM((NCHUNK, ECH), jnp.int32),
        pltpu.VMEM((ECH, 16), jnp.float32),
        pltpu.SemaphoreType.DMA,
    ],
)
def _deg_kernel(src3, dst3, ones_hbm, zeros_hbm, out,
                acc_out, acc_in, sidx, didx, ones_v, ssem):
  c = lax.axis_index("c")
  s = lax.axis_index("s")
  w = c * NS + s
  pltpu.sync_copy(zeros_hbm, acc_out.at[pl.ds(s * RT, RT)])
  pltpu.sync_copy(zeros_hbm, acc_in.at[pl.ds(s * RT, RT)])
  pltpu.sync_copy(ones_hbm, ones_v)
  pltpu.sync_copy(src3.at[w], sidx)
  pltpu.sync_copy(dst3.at[w], didx)
  plsc.subcore_barrier()
  descrs = []
  for j in range(NCHUNK):
    descrs.append(
        pltpu.async_copy(ones_v, acc_out.at[sidx.at[j]], ssem, add=True))
    descrs.append(
        pltpu.async_copy(ones_v, acc_in.at[didx.at[j]], ssem, add=True))
  for d in descrs:
    d.wait()
  plsc.subcore_barrier()
  pltpu.sync_copy(acc_out.at[pl.ds(s * RT, RT)],
                  out.at[c, 0, pl.ds(s * RT, RT)])
  pltpu.sync_copy(acc_in.at[pl.ds(s * RT, RT)],
                  out.at[c, 1, pl.ds(s * RT, RT)])


@functools.partial(
    pl.kernel,
    out_type=_f32(NC, NP, H),
    mesh=_mesh,
    compiler_params=_sc_params,
    scratch_types=[
        pltpu.VMEM_SHARED((NP, H), jnp.float32),
        pltpu.VMEM((NCHUNK, ECH), jnp.int32),
        pltpu.VMEM((NCHUNK, ECH), jnp.int32),
        pltpu.VMEM((ECH, H), jnp.float32),
        pltpu.VMEM((ECH, H), jnp.float32),
        pltpu.SemaphoreType.DMA,
        pltpu.SemaphoreType.DMA,
        pltpu.SemaphoreType.DMA,
    ],
)
def _agg_kernel(m_hbm, src3, dst3, zeros_hbm, out,
                acc, sidx, didx, rows0, rows1, g0, g1, ssem):
  c = lax.axis_index("c")
  s = lax.axis_index("s")
  w = c * NS + s
  pltpu.sync_copy(zeros_hbm, acc.at[pl.ds(s * RT, RT)])
  pltpu.sync_copy(src3.at[w], sidx)
  pltpu.sync_copy(dst3.at[w], didx)
  plsc.subcore_barrier()
  rows = (rows0, rows1)
  gsem = (g0, g1)
  gat = [pltpu.async_copy(m_hbm.at[sidx.at[0]], rows[0], gsem[0])]
  scat = []
  for j in range(NCHUNK):
    b = j & 1
    gat[j].wait()
    scat.append(pltpu.async_copy(rows[b], acc.at[didx.at[j]], ssem, add=True))
    if j + 1 < NCHUNK:
      if j >= 1:
        scat[j - 1].wait()
      gat.append(pltpu.async_copy(
          m_hbm.at[sidx.at[j + 1]], rows[(j + 1) & 1], gsem[(j + 1) & 1]))
  scat[NCHUNK - 2].wait()
  scat[NCHUNK - 1].wait()
  plsc.subcore_barrier()
  pltpu.sync_copy(acc.at[pl.ds(s * RT, RT)], out.at[c, pl.ds(s * RT, RT)])


@functools.partial(
    pl.kernel,
    out_type=[_f32(NC, G, H), _f32(NC, NS, G, H)],
    mesh=_mesh,
    compiler_params=_sc_params,
    scratch_types=[
        pltpu.VMEM_SHARED((G, H), jnp.float32),
        pltpu.VMEM((G, H), jnp.float32),
        pltpu.VMEM((NODE_W, H), jnp.float32),
        pltpu.VMEM((NODE_W, H), jnp.float32),
        pltpu.VMEM((NGC, 16), jnp.int32),
        pltpu.SemaphoreType.DMA,
    ],
)
def _readout_kernel(hw_hbm, hm_hbm, gid3, zeros_hbm, neginf_hbm,
                    out_sum, out_max, acc_sum, tbl, hw_v, hm_v, gid2, ssem):
  c = lax.axis_index("c")
  s = lax.axis_index("s")
  w = c * NS + s
  pltpu.sync_copy(zeros_hbm, acc_sum.at[pl.ds(s * GS, GS)])
  pltpu.sync_copy(neginf_hbm, tbl)
  base = w * NODE_W
  pltpu.sync_copy(hw_hbm.at[pl.ds(base, NODE_W)], hw_v)
  pltpu.sync_copy(hm_hbm.at[pl.ds(base, NODE_W)], hm_v)
  pltpu.sync_copy(gid3.at[w], gid2)
  plsc.subcore_barrier()
  descrs = []
  for ch in range(NGC):
    descrs.append(pltpu.async_copy(
        hw_v.at[pl.ds(ch * 16, 16)], acc_sum.at[gid2.at[ch]], ssem, add=True))
  lanes = lax.iota(jnp.int32, 16)

  def chunk_body(ch, carry):
    gvec = gid2[ch, :]
    for j in range(16):
      gj = jnp.max(jnp.where(lanes == j, gvec, 0))
      rowidx = jnp.full((16,), gj, jnp.int32)
      node = ch * 16 + j
      for k in range(4):
        cols = lanes + (16 * k)
        cur = plsc.load_gather(tbl, [rowidx, cols])
        hv = hm_v[node, pl.ds(k * 16, 16)]
        plsc.store_scatter(tbl, [rowidx, cols], jnp.maximum(cur, hv))
    return carry

  lax.fori_loop(0, NGC, chunk_body, 0)
  for d in descrs:
    d.wait()
  pltpu.sync_copy(tbl, out_max.at[c, s])
  plsc.subcore_barrier()
  pltpu.sync_copy(acc_sum.at[pl.ds(s * GS, GS)],
                  out_sum.at[c, pl.ds(s * GS, GS)])


def _dot(a, b):
  return jnp.dot(a, b, preferred_element_type=jnp.float32,
                 precision=lax.Precision.HIGHEST)


def _dense1_body(dp_ref, x_ref, w1_ref, wr1_ref, br1_ref,
                 m1_ref, r1_ref, no_ref, ni_ref):
  dp = dp_ref[...]
  deg_out = dp[0, 0, :, 0] + dp[1, 0, :, 0]
  deg_in = dp[0, 1, :, 0] + dp[1, 1, :, 0]
  no = lax.rsqrt(jnp.maximum(deg_out, 1.0))[:, None]
  ni = lax.rsqrt(jnp.maximum(deg_in, 1.0))[:, None]
  x = x_ref[...]
  i = pl.program_id(0)
  rows = lax.broadcasted_iota(jnp.int32, (BLK, 1), 0) + i * BLK
  m1 = _dot(x, w1_ref[...]) * no
  m1_ref[...] = jnp.where(rows < N, m1, 0.0)
  r1_ref[...] = jax.nn.relu(_dot(x, wr1_ref[...]) + br1_ref[...])
  no_ref[...] = jnp.broadcast_to(no, (BLK, H))
  ni_ref[...] = jnp.broadcast_to(ni, (BLK, H))


def _dense2_body(ap_ref, ni_ref, no_ref, r1_ref, b1_ref, w2_ref, wr2_ref,
                 br2_ref, m2_ref, r2_ref):
  agg = ap_ref[0] + ap_ref[1]
  h1 = jax.nn.relu(agg * ni_ref[...] + b1_ref[...]) + r1_ref[...]
  i = pl.program_id(0)
  rows = lax.broadcasted_iota(jnp.int32, (BLK, 1), 0) + i * BLK
  m2 = _dot(h1, w2_ref[...]) * no_ref[...]
  m2_ref[...] = jnp.where(rows < N, m2, 0.0)
  r2_ref[...] = jax.nn.relu(_dot(h1, wr2_ref[...]) + br2_ref[...])


def _dense3_body(ap_ref, ni_ref, r2_ref, b2_ref, aww_ref, awb_ref,
                 h2m_ref, hw_ref):
  agg = ap_ref[0] + ap_ref[1]
  h2 = jax.nn.relu(agg * ni_ref[...] + b2_ref[...]) + r2_ref[...]
  logit = jnp.sum(h2 * aww_ref[...], axis=1, keepdims=True) + awb_ref[0, 0]
  wgt = jax.nn.sigmoid(logit)
  i = pl.program_id(0)
  rows = lax.broadcasted_iota(jnp.int32, (BLK, 1), 0) + i * BLK
  h2m_ref[...] = jnp.where(rows < N, h2, -jnp.inf)
  hw_ref[...] = jnp.where(rows < N, h2 * wgt, 0.0)


def _head_body(sp_ref, mt_ref, w3_ref, b3_ref, w4_ref, b4_ref, out_ref):
  gsum = sp_ref[0] + sp_ref[1]
  gmax = jnp.max(mt_ref[...], axis=(0, 1))
  gmax = jnp.where(jnp.isfinite(gmax), gmax, 0.0)
  hg = jnp.concatenate([gsum, gmax], axis=1)
  z = jax.nn.relu(_dot(hg, w3_ref[...]) + b3_ref[...])
  out_ref[...] = _dot(z, w4_ref[...]) + b4_ref[...]


def _full(shape):
  return pl.BlockSpec(shape, lambda i: tuple(0 for _ in shape))


_dense1 = pl.pallas_call(
    _dense1_body,
    grid=(NP // BLK,),
    in_specs=[
        pl.BlockSpec((2, 2, BLK, 16), lambda i: (0, 0, i, 0)),
        pl.BlockSpec((BLK, DIN), lambda i: (i, 0)),
        _full((DIN, H)),
        _full((DIN, H)),
        _full((1, H)),
    ],
    out_specs=[
        pl.BlockSpec((BLK, H), lambda i: (i, 0)),
        pl.BlockSpec((BLK, H), lambda i: (i, 0)),
        pl.BlockSpec((BLK, H), lambda i: (i, 0)),
        pl.BlockSpec((BLK, H), lambda i: (i, 0)),
    ],
    out_shape=[_f32(NP, H), _f32(NP, H), _f32(NP, H), _f32(NP, H)],
)

_dense2 = pl.pallas_call(
    _dense2_body,
    grid=(NP // BLK,),
    in_specs=[
        pl.BlockSpec((2, BLK, H), lambda i: (0, i, 0)),
        pl.BlockSpec((BLK, H), lambda i: (i, 0)),
        pl.BlockSpec((BLK, H), lambda i: (i, 0)),
        pl.BlockSpec((BLK, H), lambda i: (i, 0)),
        _full((1, H)),
        _full((H, H)),
        _full((H, H)),
        _full((1, H)),
    ],
    out_specs=[
        pl.BlockSpec((BLK, H), lambda i: (i, 0)),
        pl.BlockSpec((BLK, H), lambda i: (i, 0)),
    ],
    out_shape=[_f32(NP, H), _f32(NP, H)],
)

_dense3 = pl.pallas_call(
    _dense3_body,
    grid=(NP // BLK,),
    in_specs=[
        pl.BlockSpec((2, BLK, H), lambda i: (0, i, 0)),
        pl.BlockSpec((BLK, H), lambda i: (i, 0)),
        pl.BlockSpec((BLK, H), lambda i: (i, 0)),
        _full((1, H)),
        _full((1, H)),
        _full((1, 1)),
    ],
    out_specs=[
        pl.BlockSpec((BLK, H), lambda i: (i, 0)),
        pl.BlockSpec((BLK, H), lambda i: (i, 0)),
    ],
    out_shape=[_f32(NP, H), _f32(NP, H)],
)

_head = pl.pallas_call(
    _head_body,
    grid=(1,),
    in_specs=[
        _full((NC, G, H)),
        _full((NC, NS, G, H)),
        _full((2 * H, DIM)),
        _full((1, DIM)),
        _full((DIM, DIM)),
        _full((1, DIM)),
    ],
    out_specs=_full((G, DIM)),
    out_shape=_f32(G, DIM),
)


def kernel(x, efeat, edge_index, graph_ids, params):
  del efeat
  f32 = jnp.float32
  xp = jnp.concatenate([x, jnp.zeros((NP - N, DIN), f32)], axis=0)
  pad = jnp.full((EP - E,), NP - 1, jnp.int32)
  src3 = jnp.concatenate([edge_index[0], pad]).reshape(NW, NCHUNK, ECH)
  dst3 = jnp.concatenate([edge_index[1], pad]).reshape(NW, NCHUNK, ECH)
  gid3 = jnp.concatenate(
      [graph_ids, jnp.full((NP - N,), G - 1, jnp.int32)]).reshape(NW, NGC, 16)

  ones_ech16 = jnp.ones((ECH, 16), f32)
  zeros_rt16 = jnp.zeros((RT, 16), f32)
  zeros_rth = jnp.zeros((RT, H), f32)
  zeros_gsh = jnp.zeros((GS, H), f32)
  neginf_gh = jnp.full((G, H), -jnp.inf, f32)

  p1, p2 = params['layer1'], params['layer2']

  deg_parts = _deg_kernel(src3, dst3, ones_ech16, zeros_rt16)
  m1, r1, no64, ni64 = _dense1(deg_parts, xp, p1['W'], p1['Wr'],
                               p1['br'].reshape(1, H))
  agg1 = _agg_kernel(m1, src3, dst3, zeros_rth)
  m2, r2 = _dense2(agg1, ni64, no64, r1, p1['b'].reshape(1, H),
                   p2['W'], p2['Wr'], p2['br'].reshape(1, H))
  agg2 = _agg_kernel(m2, src3, dst3, zeros_rth)
  h2m, hw = _dense3(agg2, ni64, r2, p2['b'].reshape(1, H),
                    params['aw_w'].reshape(1, H),
                    params['aw_b'].reshape(1, 1))
  sum_parts, max_tbls = _readout_kernel(hw, h2m, gid3, zeros_gsh, neginf_gh)
  latent = _head(sum_parts, max_tbls, params['W3'],
                 params['b3'].reshape(1, DIM), params['W4'],
                 params['b4'].reshape(1, DIM))
  zero = jnp.zeros((1, DIM), f32)
  for idx in (10, 100, 300):
    latent = jnp.concatenate([latent[:idx], zero, latent[idx:]], axis=0)
  return latent


# speedup vs baseline: 7.8442x; 1.0972x over previous
import functools

import jax
import jax.numpy as jnp
from jax import lax
from jax.experimental import pallas as pl
from jax.experimental.pallas import tpu as pltpu
from jax.experimental.pallas import tpu_sc as plsc

N = 10000
NP = 10240
E = 320000
DIN = 128
H = 64
DIM = 128
G = 512
NC = 2
NS = 16
NW = NC * NS
ECH = 128
NCHUNK = 79
EW = ECH * NCHUNK
EP = EW * NW
RT = NP // NS
NODE_W = NP // NW
NGC = NODE_W // 16
GS = G // NS
BLK = 1024
NBUF = 6
PF = 3

_mesh = plsc.VectorSubcoreMesh(
    core_axis_name="c", subcore_axis_name="s", num_cores=NC, num_subcores=NS)
_sc_params = pltpu.CompilerParams(
    use_tc_tiling_on_sc=False, needs_layout_passes=False)


def _f32(*shape):
  return jax.ShapeDtypeStruct(shape, jnp.float32)


@functools.partial(
    pl.kernel,
    out_type=_f32(NC, 2, NP, 16),
    mesh=_mesh,
    compiler_params=_sc_params,
    scratch_types=[
        pltpu.VMEM_SHARED((NP, 16), jnp.float32),
        pltpu.VMEM_SHARED((NP, 16), jnp.float32),
        pltpu.VMEM((NCHUNK, ECH), jnp.int32),
        pltpu.VMEM((NCHUNK, ECH), jnp.int32),
        pltpu.VMEM((ECH, 16), jnp.float32),
        pltpu.SemaphoreType.DMA,
    ],
)
def _deg_kernel(src3, dst3, ones_hbm, zeros_hbm, out,
                acc_out, acc_in, sidx, didx, ones_v, ssem):
  c = lax.axis_index("c")
  s = lax.axis_index("s")
  w = c * NS + s
  pltpu.sync_copy(zeros_hbm, acc_out.at[pl.ds(s * RT, RT)])
  pltpu.sync_copy(zeros_hbm, acc_in.at[pl.ds(s * RT, RT)])
  pltpu.sync_copy(ones_hbm, ones_v)
  pltpu.sync_copy(src3.at[w], sidx)
  pltpu.sync_copy(dst3.at[w], didx)
  plsc.subcore_barrier()
  descrs = []
  for j in range(NCHUNK):
    descrs.append(
        pltpu.async_copy(ones_v, acc_out.at[sidx.at[j]], ssem, add=True))
    descrs.append(
        pltpu.async_copy(ones_v, acc_in.at[didx.at[j]], ssem, add=True))
  for d in descrs:
    d.wait()
  plsc.subcore_barrier()
  pltpu.sync_copy(acc_out.at[pl.ds(s * RT, RT)],
                  out.at[c, 0, pl.ds(s * RT, RT)])
  pltpu.sync_copy(acc_in.at[pl.ds(s * RT, RT)],
                  out.at[c, 1, pl.ds(s * RT, RT)])


@functools.partial(
    pl.kernel,
    out_type=_f32(NC, NP, H),
    mesh=_mesh,
    compiler_params=_sc_params,
    scratch_types=[
        pltpu.VMEM_SHARED((NP, H), jnp.float32),
        pltpu.VMEM((NCHUNK, ECH), jnp.int32),
        pltpu.VMEM((NCHUNK, ECH), jnp.int32),
        pltpu.VMEM((NBUF, ECH, H), jnp.float32),
        [pltpu.SemaphoreType.DMA] * NBUF,
        pltpu.SemaphoreType.DMA,
    ],
)
def _agg_kernel(m_hbm, src3, dst3, zeros_hbm, out,
                acc, sidx, didx, rows, gsem, ssem):
  c = lax.axis_index("c")
  s = lax.axis_index("s")
  w = c * NS + s
  pltpu.sync_copy(zeros_hbm, acc.at[pl.ds(s * RT, RT)])
  pltpu.sync_copy(src3.at[w], sidx)
  pltpu.sync_copy(dst3.at[w], didx)
  plsc.subcore_barrier()
  gat, scat = [], []
  for j in range(PF):
    gat.append(pltpu.async_copy(
        m_hbm.at[sidx.at[j]], rows.at[j % NBUF], gsem[j % NBUF]))
  for j in range(NCHUNK):
    gat[j].wait()
    scat.append(pltpu.async_copy(
        rows.at[j % NBUF], acc.at[didx.at[j]], ssem, add=True))
    jn = j + PF
    if jn < NCHUNK:
      if jn >= NBUF:
        scat[jn - NBUF].wait()
      gat.append(pltpu.async_copy(
          m_hbm.at[sidx.at[jn]], rows.at[jn % NBUF], gsem[jn % NBUF]))
  for j in range(max(0, NCHUNK - NBUF), NCHUNK):
    scat[j].wait()
  plsc.subcore_barrier()
  pltpu.sync_copy(acc.at[pl.ds(s * RT, RT)], out.at[c, pl.ds(s * RT, RT)])


@functools.partial(
    pl.kernel,
    out_type=[_f32(NC, G, H), _f32(NC, NS, G, H)],
    mesh=_mesh,
    compiler_params=_sc_params,
    scratch_types=[
        pltpu.VMEM_SHARED((G, H), jnp.float32),
        pltpu.VMEM((G, H), jnp.float32),
        pltpu.VMEM((NODE_W, H), jnp.float32),
        pltpu.VMEM((NODE_W, H), jnp.float32),
        pltpu.VMEM((NGC, 16), jnp.int32),
        pltpu.SemaphoreType.DMA,
    ],
)
def _readout_kernel(hw_hbm, hm_hbm, gid3, zeros_hbm, neginf_hbm,
                    out_sum, out_max, acc_sum, tbl, hw_v, hm_v, gid2, ssem):
  c = lax.axis_index("c")
  s = lax.axis_index("s")
  w = c * NS + s
  pltpu.sync_copy(zeros_hbm, acc_sum.at[pl.ds(s * GS, GS)])
  pltpu.sync_copy(neginf_hbm, tbl)
  base = w * NODE_W
  pltpu.sync_copy(hw_hbm.at[pl.ds(base, NODE_W)], hw_v)
  pltpu.sync_copy(hm_hbm.at[pl.ds(base, NODE_W)], hm_v)
  pltpu.sync_copy(gid3.at[w], gid2)
  plsc.subcore_barrier()
  descrs = []
  for ch in range(NGC):
    descrs.append(pltpu.async_copy(
        hw_v.at[pl.ds(ch * 16, 16)], acc_sum.at[gid2.at[ch]], ssem, add=True))
  lanes = lax.iota(jnp.int32, 16)

  def chunk_body(ch, carry):
    gvec = gid2[ch, :]
    for j in range(16):
      gj = jnp.max(jnp.where(lanes == j, gvec, 0))
      rowidx = jnp.full((16,), gj, jnp.int32)
      node = ch * 16 + j
      for k in range(4):
        cols = lanes + (16 * k)
        cur = plsc.load_gather(tbl, [rowidx, cols])
        hv = hm_v[node, pl.ds(k * 16, 16)]
        plsc.store_scatter(tbl, [rowidx, cols], jnp.maximum(cur, hv))
    return carry

  lax.fori_loop(0, NGC, chunk_body, 0)
  for d in descrs:
    d.wait()
  pltpu.sync_copy(tbl, out_max.at[c, s])
  plsc.subcore_barrier()
  pltpu.sync_copy(acc_sum.at[pl.ds(s * GS, GS)],
                  out_sum.at[c, pl.ds(s * GS, GS)])


def _dot(a, b):
  return jnp.dot(a, b, preferred_element_type=jnp.float32,
                 precision=lax.Precision.HIGHEST)


def _dense1_body(dp_ref, x_ref, w1_ref, wr1_ref, br1_ref,
                 m1_ref, r1_ref, no_ref, ni_ref):
  dp = dp_ref[...]
  deg_out = dp[0, 0, :, 0] + dp[1, 0, :, 0]
  deg_in = dp[0, 1, :, 0] + dp[1, 1, :, 0]
  no = lax.rsqrt(jnp.maximum(deg_out, 1.0))[:, None]
  ni = lax.rsqrt(jnp.maximum(deg_in, 1.0))[:, None]
  x = x_ref[...]
  i = pl.program_id(0)
  rows = lax.broadcasted_iota(jnp.int32, (BLK, 1), 0) + i * BLK
  m1 = _dot(x, w1_ref[...]) * no
  m1_ref[...] = jnp.where(rows < N, m1, 0.0)
  r1_ref[...] = jax.nn.relu(_dot(x, wr1_ref[...]) + br1_ref[...])
  no_ref[...] = jnp.broadcast_to(no, (BLK, H))
  ni_ref[...] = jnp.broadcast_to(ni, (BLK, H))


def _dense2_body(ap_ref, ni_ref, no_ref, r1_ref, b1_ref, w2_ref, wr2_ref,
                 br2_ref, m2_ref, r2_ref):
  agg = ap_ref[0] + ap_ref[1]
  h1 = jax.nn.relu(agg * ni_ref[...] + b1_ref[...]) + r1_ref[...]
  i = pl.program_id(0)
  rows = lax.broadcasted_iota(jnp.int32, (BLK, 1), 0) + i * BLK
  m2 = _dot(h1, w2_ref[...]) * no_ref[...]
  m2_ref[...] = jnp.where(rows < N, m2, 0.0)
  r2_ref[...] = jax.nn.relu(_dot(h1, wr2_ref[...]) + br2_ref[...])


def _dense3_body(ap_ref, ni_ref, r2_ref, b2_ref, aww_ref, awb_ref,
                 h2m_ref, hw_ref):
  agg = ap_ref[0] + ap_ref[1]
  h2 = jax.nn.relu(agg * ni_ref[...] + b2_ref[...]) + r2_ref[...]
  logit = jnp.sum(h2 * aww_ref[...], axis=1, keepdims=True) + awb_ref[0, 0]
  wgt = jax.nn.sigmoid(logit)
  i = pl.program_id(0)
  rows = lax.broadcasted_iota(jnp.int32, (BLK, 1), 0) + i * BLK
  h2m_ref[...] = jnp.where(rows < N, h2, -jnp.inf)
  hw_ref[...] = jnp.where(rows < N, h2 * wgt, 0.0)


def _head_body(sp_ref, mt_ref, w3_ref, b3_ref, w4_ref, b4_ref, out_ref):
  gsum = sp_ref[0] + sp_ref[1]
  gmax = jnp.max(mt_ref[...], axis=(0, 1))
  gmax = jnp.where(jnp.isfinite(gmax), gmax, 0.0)
  hg = jnp.concatenate([gsum, gmax], axis=1)
  z = jax.nn.relu(_dot(hg, w3_ref[...]) + b3_ref[...])
  out_ref[...] = _dot(z, w4_ref[...]) + b4_ref[...]


def _full(shape):
  return pl.BlockSpec(shape, lambda i: tuple(0 for _ in shape))


_dense1 = pl.pallas_call(
    _dense1_body,
    grid=(NP // BLK,),
    in_specs=[
        pl.BlockSpec((2, 2, BLK, 16), lambda i: (0, 0, i, 0)),
        pl.BlockSpec((BLK, DIN), lambda i: (i, 0)),
        _full((DIN, H)),
        _full((DIN, H)),
        _full((1, H)),
    ],
    out_specs=[
        pl.BlockSpec((BLK, H), lambda i: (i, 0)),
        pl.BlockSpec((BLK, H), lambda i: (i, 0)),
        pl.BlockSpec((BLK, H), lambda i: (i, 0)),
        pl.BlockSpec((BLK, H), lambda i: (i, 0)),
    ],
    out_shape=[_f32(NP, H), _f32(NP, H), _f32(NP, H), _f32(NP, H)],
)

_dense2 = pl.pallas_call(
    _dense2_body,
    grid=(NP // BLK,),
    in_specs=[
        pl.BlockSpec((2, BLK, H), lambda i: (0, i, 0)),
        pl.BlockSpec((BLK, H), lambda i: (i, 0)),
        pl.BlockSpec((BLK, H), lambda i: (i, 0)),
        pl.BlockSpec((BLK, H), lambda i: (i, 0)),
        _full((1, H)),
        _full((H, H)),
        _full((H, H)),
        _full((1, H)),
    ],
    out_specs=[
        pl.BlockSpec((BLK, H), lambda i: (i, 0)),
        pl.BlockSpec((BLK, H), lambda i: (i, 0)),
    ],
    out_shape=[_f32(NP, H), _f32(NP, H)],
)

_dense3 = pl.pallas_call(
    _dense3_body,
    grid=(NP // BLK,),
    in_specs=[
        pl.BlockSpec((2, BLK, H), lambda i: (0, i, 0)),
        pl.BlockSpec((BLK, H), lambda i: (i, 0)),
        pl.BlockSpec((BLK, H), lambda i: (i, 0)),
        _full((1, H)),
        _full((1, H)),
        _full((1, 1)),
    ],
    out_specs=[
        pl.BlockSpec((BLK, H), lambda i: (i, 0)),
        pl.BlockSpec((BLK, H), lambda i: (i, 0)),
    ],
    out_shape=[_f32(NP, H), _f32(NP, H)],
)

_head = pl.pallas_call(
    _head_body,
    grid=(1,),
    in_specs=[
        _full((NC, G, H)),
        _full((NC, NS, G, H)),
        _full((2 * H, DIM)),
        _full((1, DIM)),
        _full((DIM, DIM)),
        _full((1, DIM)),
    ],
    out_specs=_full((G, DIM)),
    out_shape=_f32(G, DIM),
)


def kernel(x, efeat, edge_index, graph_ids, params):
  del efeat
  f32 = jnp.float32
  xp = jnp.concatenate([x, jnp.zeros((NP - N, DIN), f32)], axis=0)
  pad = jnp.full((EP - E,), NP - 1, jnp.int32)
  src3 = jnp.concatenate([edge_index[0], pad]).reshape(NW, NCHUNK, ECH)
  dst3 = jnp.concatenate([edge_index[1], pad]).reshape(NW, NCHUNK, ECH)
  gid3 = jnp.concatenate(
      [graph_ids, jnp.full((NP - N,), G - 1, jnp.int32)]).reshape(NW, NGC, 16)

  ones_ech16 = jnp.ones((ECH, 16), f32)
  zeros_rt16 = jnp.zeros((RT, 16), f32)
  zeros_rth = jnp.zeros((RT, H), f32)
  zeros_gsh = jnp.zeros((GS, H), f32)
  neginf_gh = jnp.full((G, H), -jnp.inf, f32)

  p1, p2 = params['layer1'], params['layer2']

  deg_parts = _deg_kernel(src3, dst3, ones_ech16, zeros_rt16)
  m1, r1, no64, ni64 = _dense1(deg_parts, xp, p1['W'], p1['Wr'],
                               p1['br'].reshape(1, H))
  agg1 = _agg_kernel(m1, src3, dst3, zeros_rth)
  m2, r2 = _dense2(agg1, ni64, no64, r1, p1['b'].reshape(1, H),
                   p2['W'], p2['Wr'], p2['br'].reshape(1, H))
  agg2 = _agg_kernel(m2, src3, dst3, zeros_rth)
  h2m, hw = _dense3(agg2, ni64, r2, p2['b'].reshape(1, H),
                    params['aw_w'].reshape(1, H),
                    params['aw_b'].reshape(1, 1))
  sum_parts, max_tbls = _readout_kernel(hw, h2m, gid3, zeros_gsh, neginf_gh)
  latent = _head(sum_parts, max_tbls, params['W3'],
                 params['b3'].reshape(1, DIM), params['W4'],
                 params['b4'].reshape(1, DIM))
  zero = jnp.zeros((1, DIM), f32)
  for idx in (10, 100, 300):
    latent = jnp.concatenate([latent[:idx], zero, latent[idx:]], axis=0)
  return latent


# speedup vs baseline: 10.6513x; 1.3578x over previous
import functools

import jax
import jax.numpy as jnp
from jax import lax
from jax.experimental import pallas as pl
from jax.experimental.pallas import tpu as pltpu
from jax.experimental.pallas import tpu_sc as plsc

N = 10000
NP = 10240
E = 320000
DIN = 128
H = 64
DIM = 128
G = 512
NC = 2
NS = 16
NW = NC * NS
ECH = 125
TCH = E // ECH
NCHUNK = TCH // NW
K0 = 111
K1 = 49
RT = NP // NS
NODE_W = NP // NW
NGC = NODE_W // 16
GS = G // NS
BLK = 1024
NBUF = 6
PF = 3

_mesh = plsc.VectorSubcoreMesh(
    core_axis_name="c", subcore_axis_name="s", num_cores=NC, num_subcores=NS)
_sc_params = pltpu.CompilerParams(
    use_tc_tiling_on_sc=False, needs_layout_passes=False)


def _f32(*shape):
  return jax.ShapeDtypeStruct(shape, jnp.float32)


@functools.partial(
    pl.kernel,
    out_type=_f32(NC, 2, NP, 16),
    mesh=_mesh,
    compiler_params=_sc_params,
    scratch_types=[
        pltpu.VMEM_SHARED((NP, 16), jnp.float32),
        pltpu.VMEM_SHARED((NP, 16), jnp.float32),
        pltpu.VMEM((NCHUNK, ECH), jnp.int32),
        pltpu.VMEM((NCHUNK, ECH), jnp.int32),
        pltpu.VMEM((ECH, 16), jnp.float32),
        pltpu.SemaphoreType.DMA,
    ],
)
def _deg_kernel(src2, dst2, ones_hbm, zeros_hbm, out,
                acc_out, acc_in, sidx, didx, ones_v, ssem):
  c = lax.axis_index("c")
  s = lax.axis_index("s")
  w = c * NS + s
  pltpu.sync_copy(zeros_hbm, acc_out.at[pl.ds(s * RT, RT)])
  pltpu.sync_copy(zeros_hbm, acc_in.at[pl.ds(s * RT, RT)])
  pltpu.sync_copy(ones_hbm, ones_v)
  pltpu.sync_copy(src2.at[pl.ds(w * NCHUNK, NCHUNK)], sidx)
  pltpu.sync_copy(dst2.at[pl.ds(w * NCHUNK, NCHUNK)], didx)
  plsc.subcore_barrier()
  descrs = []
  for j in range(NCHUNK):
    descrs.append(
        pltpu.async_copy(ones_v, acc_out.at[sidx.at[j]], ssem, add=True))
    descrs.append(
        pltpu.async_copy(ones_v, acc_in.at[didx.at[j]], ssem, add=True))
  for d in descrs:
    d.wait()
  plsc.subcore_barrier()
  pltpu.sync_copy(acc_out.at[pl.ds(s * RT, RT)],
                  out.at[c, 0, pl.ds(s * RT, RT)])
  pltpu.sync_copy(acc_in.at[pl.ds(s * RT, RT)],
                  out.at[c, 1, pl.ds(s * RT, RT)])


@functools.partial(
    pl.kernel,
    out_type=_f32(NC, NP, H),
    mesh=_mesh,
    compiler_params=_sc_params,
    scratch_types=[
        pltpu.VMEM_SHARED((NP, H), jnp.float32),
        pltpu.VMEM((K0, ECH), jnp.int32),
        pltpu.VMEM((K0, ECH), jnp.int32),
        pltpu.VMEM((NBUF, ECH, H), jnp.float32),
        [pltpu.SemaphoreType.DMA] * NBUF,
        pltpu.SemaphoreType.DMA,
    ],
)
def _agg_kernel(m_hbm, src2, dst2, zeros_hbm, out,
                acc, sidx, didx, rows, gsem, ssem):
  c = lax.axis_index("c")
  s = lax.axis_index("s")
  pltpu.sync_copy(zeros_hbm, acc.at[pl.ds(s * RT, RT)])
  plsc.subcore_barrier()

  def pipeline(base, k):
    pltpu.sync_copy(src2.at[pl.ds(base, k)], sidx.at[pl.ds(0, k)])
    pltpu.sync_copy(dst2.at[pl.ds(base, k)], didx.at[pl.ds(0, k)])
    gat, scat = [], []
    for j in range(min(PF, k)):
      gat.append(pltpu.async_copy(
          m_hbm.at[sidx.at[j]], rows.at[j % NBUF], gsem[j % NBUF]))
    for j in range(k):
      gat[j].wait()
      scat.append(pltpu.async_copy(
          rows.at[j % NBUF], acc.at[didx.at[j]], ssem, add=True))
      jn = j + PF
      if jn < k:
        if jn >= NBUF:
          scat[jn - NBUF].wait()
        gat.append(pltpu.async_copy(
            m_hbm.at[sidx.at[jn]], rows.at[jn % NBUF], gsem[jn % NBUF]))
    for j in range(max(0, k - NBUF), k):
      scat[j].wait()

  @pl.when(c == 0)
  def _():
    pipeline(s * K0, K0)

  @pl.when(c == 1)
  def _():
    pipeline(NS * K0 + s * K1, K1)

  plsc.subcore_barrier()
  pltpu.sync_copy(acc.at[pl.ds(s * RT, RT)], out.at[c, pl.ds(s * RT, RT)])


@functools.partial(
    pl.kernel,
    out_type=[_f32(NC, G, H), _f32(NC, NS, G, H)],
    mesh=_mesh,
    compiler_params=_sc_params,
    scratch_types=[
        pltpu.VMEM_SHARED((G, H), jnp.float32),
        pltpu.VMEM((G, H), jnp.float32),
        pltpu.VMEM((NODE_W, H), jnp.float32),
        pltpu.VMEM((NODE_W, H), jnp.float32),
        pltpu.VMEM((NGC, 16), jnp.int32),
        pltpu.SemaphoreType.DMA,
    ],
)
def _readout_kernel(hw_hbm, hm_hbm, gid3, zeros_hbm, neginf_hbm,
                    out_sum, out_max, acc_sum, tbl, hw_v, hm_v, gid2, ssem):
  c = lax.axis_index("c")
  s = lax.axis_index("s")
  w = c * NS + s
  pltpu.sync_copy(zeros_hbm, acc_sum.at[pl.ds(s * GS, GS)])
  pltpu.sync_copy(neginf_hbm, tbl)
  base = w * NODE_W
  pltpu.sync_copy(hw_hbm.at[pl.ds(base, NODE_W)], hw_v)
  pltpu.sync_copy(hm_hbm.at[pl.ds(base, NODE_W)], hm_v)
  pltpu.sync_copy(gid3.at[w], gid2)
  plsc.subcore_barrier()
  descrs = []
  for ch in range(NGC):
    descrs.append(pltpu.async_copy(
        hw_v.at[pl.ds(ch * 16, 16)], acc_sum.at[gid2.at[ch]], ssem, add=True))
  lanes = lax.iota(jnp.int32, 16)

  def chunk_body(ch, carry):
    gvec = gid2[ch, :]
    for j in range(16):
      gj = jnp.max(jnp.where(lanes == j, gvec, 0))
      rowidx = jnp.full((16,), gj, jnp.int32)
      node = ch * 16 + j
      for k in range(4):
        cols = lanes + (16 * k)
        cur = plsc.load_gather(tbl, [rowidx, cols])
        hv = hm_v[node, pl.ds(k * 16, 16)]
        plsc.store_scatter(tbl, [rowidx, cols], jnp.maximum(cur, hv))
    return carry

  lax.fori_loop(0, NGC, chunk_body, 0)
  for d in descrs:
    d.wait()
  pltpu.sync_copy(tbl, out_max.at[c, s])
  plsc.subcore_barrier()
  pltpu.sync_copy(acc_sum.at[pl.ds(s * GS, GS)],
                  out_sum.at[c, pl.ds(s * GS, GS)])


def _dot(a, b):
  return jnp.dot(a, b, preferred_element_type=jnp.float32,
                 precision=lax.Precision.HIGHEST)


def _dense1_body(dp_ref, x_ref, w1_ref, wr1_ref, br1_ref,
                 m1_ref, r1_ref, no_ref, ni_ref):
  dp = dp_ref[...]
  deg_out = dp[0, 0, :, 0] + dp[1, 0, :, 0]
  deg_in = dp[0, 1, :, 0] + dp[1, 1, :, 0]
  no = lax.rsqrt(jnp.maximum(deg_out, 1.0))[:, None]
  ni = lax.rsqrt(jnp.maximum(deg_in, 1.0))[:, None]
  x = x_ref[...]
  m1_ref[...] = _dot(x, w1_ref[...]) * no
  r1_ref[...] = jax.nn.relu(_dot(x, wr1_ref[...]) + br1_ref[...])
  no_ref[...] = jnp.broadcast_to(no, (BLK, H))
  ni_ref[...] = jnp.broadcast_to(ni, (BLK, H))


def _dense2_body(ap_ref, ni_ref, no_ref, r1_ref, b1_ref, w2_ref, wr2_ref,
                 br2_ref, m2_ref, r2_ref):
  agg = ap_ref[0] + ap_ref[1]
  h1 = jax.nn.relu(agg * ni_ref[...] + b1_ref[...]) + r1_ref[...]
  m2_ref[...] = _dot(h1, w2_ref[...]) * no_ref[...]
  r2_ref[...] = jax.nn.relu(_dot(h1, wr2_ref[...]) + br2_ref[...])


def _dense3_body(ap_ref, ni_ref, r2_ref, b2_ref, aww_ref, awb_ref,
                 h2m_ref, hw_ref):
  agg = ap_ref[0] + ap_ref[1]
  h2 = jax.nn.relu(agg * ni_ref[...] + b2_ref[...]) + r2_ref[...]
  logit = jnp.sum(h2 * aww_ref[...], axis=1, keepdims=True) + awb_ref[0, 0]
  wgt = jax.nn.sigmoid(logit)
  i = pl.program_id(0)
  rows = lax.broadcasted_iota(jnp.int32, (BLK, 1), 0) + i * BLK
  h2m_ref[...] = jnp.where(rows < N, h2, -jnp.inf)
  hw_ref[...] = jnp.where(rows < N, h2 * wgt, 0.0)


def _head_body(sp_ref, mt_ref, w3_ref, b3_ref, w4_ref, b4_ref, out_ref):
  gsum = sp_ref[0] + sp_ref[1]
  gmax = jnp.max(mt_ref[...], axis=(0, 1))
  gmax = jnp.where(jnp.isfinite(gmax), gmax, 0.0)
  hg = jnp.concatenate([gsum, gmax], axis=1)
  z = jax.nn.relu(_dot(hg, w3_ref[...]) + b3_ref[...])
  out_ref[...] = _dot(z, w4_ref[...]) + b4_ref[...]


def _full(shape):
  return pl.BlockSpec(shape, lambda i: tuple(0 for _ in shape))


_dense1 = pl.pallas_call(
    _dense1_body,
    grid=(NP // BLK,),
    in_specs=[
        pl.BlockSpec((2, 2, BLK, 16), lambda i: (0, 0, i, 0)),
        pl.BlockSpec((BLK, DIN), lambda i: (i, 0)),
        _full((DIN, H)),
        _full((DIN, H)),
        _full((1, H)),
    ],
    out_specs=[
        pl.BlockSpec((BLK, H), lambda i: (i, 0)),
        pl.BlockSpec((BLK, H), lambda i: (i, 0)),
        pl.BlockSpec((BLK, H), lambda i: (i, 0)),
        pl.BlockSpec((BLK, H), lambda i: (i, 0)),
    ],
    out_shape=[_f32(NP, H), _f32(NP, H), _f32(NP, H), _f32(NP, H)],
)

_dense2 = pl.pallas_call(
    _dense2_body,
    grid=(NP // BLK,),
    in_specs=[
        pl.BlockSpec((2, BLK, H), lambda i: (0, i, 0)),
        pl.BlockSpec((BLK, H), lambda i: (i, 0)),
        pl.BlockSpec((BLK, H), lambda i: (i, 0)),
        pl.BlockSpec((BLK, H), lambda i: (i, 0)),
        _full((1, H)),
        _full((H, H)),
        _full((H, H)),
        _full((1, H)),
    ],
    out_specs=[
        pl.BlockSpec((BLK, H), lambda i: (i, 0)),
        pl.BlockSpec((BLK, H), lambda i: (i, 0)),
    ],
    out_shape=[_f32(NP, H), _f32(NP, H)],
)

_dense3 = pl.pallas_call(
    _dense3_body,
    grid=(NP // BLK,),
    in_specs=[
        pl.BlockSpec((2, BLK, H), lambda i: (0, i, 0)),
        pl.BlockSpec((BLK, H), lambda i: (i, 0)),
        pl.BlockSpec((BLK, H), lambda i: (i, 0)),
        _full((1, H)),
        _full((1, H)),
        _full((1, 1)),
    ],
    out_specs=[
        pl.BlockSpec((BLK, H), lambda i: (i, 0)),
        pl.BlockSpec((BLK, H), lambda i: (i, 0)),
    ],
    out_shape=[_f32(NP, H), _f32(NP, H)],
)

_head = pl.pallas_call(
    _head_body,
    grid=(1,),
    in_specs=[
        _full((NC, G, H)),
        _full((NC, NS, G, H)),
        _full((2 * H, DIM)),
        _full((1, DIM)),
        _full((DIM, DIM)),
        _full((1, DIM)),
    ],
    out_specs=_full((G, DIM)),
    out_shape=_f32(G, DIM),
)


def kernel(x, efeat, edge_index, graph_ids, params):
  del efeat
  f32 = jnp.float32
  xp = jnp.concatenate([x, jnp.zeros((NP - N, DIN), f32)], axis=0)
  src2 = edge_index[0].reshape(TCH, ECH)
  dst2 = edge_index[1].reshape(TCH, ECH)
  gid3 = jnp.concatenate(
      [graph_ids, jnp.full((NP - N,), G - 1, jnp.int32)]).reshape(NW, NGC, 16)

  ones_ech16 = jnp.ones((ECH, 16), f32)
  zeros_rt16 = jnp.zeros((RT, 16), f32)
  zeros_rth = jnp.zeros((RT, H), f32)
  zeros_gsh = jnp.zeros((GS, H), f32)
  neginf_gh = jnp.full((G, H), -jnp.inf, f32)

  p1, p2 = params['layer1'], params['layer2']

  deg_parts = _deg_kernel(src2, dst2, ones_ech16, zeros_rt16)
  m1, r1, no64, ni64 = _dense1(deg_parts, xp, p1['W'], p1['Wr'],
                               p1['br'].reshape(1, H))
  agg1 = _agg_kernel(m1, src2, dst2, zeros_rth)
  m2, r2 = _dense2(agg1, ni64, no64, r1, p1['b'].reshape(1, H),
                   p2['W'], p2['Wr'], p2['br'].reshape(1, H))
  agg2 = _agg_kernel(m2, src2, dst2, zeros_rth)
  h2m, hw = _dense3(agg2, ni64, r2, p2['b'].reshape(1, H),
                    params['aw_w'].reshape(1, H),
                    params['aw_b'].reshape(1, 1))
  sum_parts, max_tbls = _readout_kernel(hw, h2m, gid3, zeros_gsh, neginf_gh)
  latent = _head(sum_parts, max_tbls, params['W3'],
                 params['b3'].reshape(1, DIM), params['W4'],
                 params['b4'].reshape(1, DIM))
  zero = jnp.zeros((1, DIM), f32)
  for idx in (10, 100, 300):
    latent = jnp.concatenate([latent[:idx], zero, latent[idx:]], axis=0)
  return latent


# speedup vs baseline: 11.4912x; 1.0789x over previous
import functools

import jax
import jax.numpy as jnp
from jax import lax
from jax.experimental import pallas as pl
from jax.experimental.pallas import tpu as pltpu
from jax.experimental.pallas import tpu_sc as plsc

N = 10000
NP = 10240
E = 320000
DIN = 128
H = 64
DIM = 128
G = 512
NC = 2
NS = 16
NW = NC * NS
ECH = 125
TCH = E // ECH
NCHUNK = TCH // NW
K0 = 111
K1 = 49
RT = NP // NS
NODE_W = NP // NW
NGC = NODE_W // 16
GS = G // NS
BLK = 1024
NBUF = 6
PF = 4
NOUT = 515

_mesh = plsc.VectorSubcoreMesh(
    core_axis_name="c", subcore_axis_name="s", num_cores=NC, num_subcores=NS)
_sc_params = pltpu.CompilerParams(
    use_tc_tiling_on_sc=False, needs_layout_passes=False)


def _f32(*shape):
  return jax.ShapeDtypeStruct(shape, jnp.float32)


@functools.partial(
    pl.kernel,
    out_type=_f32(NC, 2, NP, 16),
    mesh=_mesh,
    compiler_params=_sc_params,
    scratch_types=[
        pltpu.VMEM_SHARED((NP, 16), jnp.float32),
        pltpu.VMEM_SHARED((NP, 16), jnp.float32),
        pltpu.VMEM((NCHUNK, ECH), jnp.int32),
        pltpu.VMEM((NCHUNK, ECH), jnp.int32),
        pltpu.VMEM((ECH, 16), jnp.float32),
        pltpu.SemaphoreType.DMA,
    ],
)
def _deg_kernel(e3, ones_hbm, zeros_hbm, out,
                acc_out, acc_in, sidx, didx, ones_v, ssem):
  c = lax.axis_index("c")
  s = lax.axis_index("s")
  w = c * NS + s
  pltpu.sync_copy(zeros_hbm, acc_out.at[pl.ds(s * RT, RT)])
  pltpu.sync_copy(zeros_hbm, acc_in.at[pl.ds(s * RT, RT)])
  pltpu.sync_copy(ones_hbm, ones_v)
  pltpu.sync_copy(e3.at[0, pl.ds(w * NCHUNK, NCHUNK)], sidx)
  pltpu.sync_copy(e3.at[1, pl.ds(w * NCHUNK, NCHUNK)], didx)
  plsc.subcore_barrier()
  descrs = []
  for j in range(NCHUNK):
    descrs.append(
        pltpu.async_copy(ones_v, acc_out.at[sidx.at[j]], ssem, add=True))
    descrs.append(
        pltpu.async_copy(ones_v, acc_in.at[didx.at[j]], ssem, add=True))
  for d in descrs:
    d.wait()
  plsc.subcore_barrier()
  pltpu.sync_copy(acc_out.at[pl.ds(s * RT, RT)],
                  out.at[c, 0, pl.ds(s * RT, RT)])
  pltpu.sync_copy(acc_in.at[pl.ds(s * RT, RT)],
                  out.at[c, 1, pl.ds(s * RT, RT)])


@functools.partial(
    pl.kernel,
    out_type=_f32(NC, NP, H),
    mesh=_mesh,
    compiler_params=_sc_params,
    scratch_types=[
        pltpu.VMEM_SHARED((NP, H), jnp.float32),
        pltpu.VMEM((K0, ECH), jnp.int32),
        pltpu.VMEM((K0, ECH), jnp.int32),
        pltpu.VMEM((NBUF, ECH, H), jnp.float32),
        [pltpu.SemaphoreType.DMA] * NBUF,
        pltpu.SemaphoreType.DMA,
    ],
)
def _agg_kernel(m_hbm, e3, zeros_hbm, out,
                acc, sidx, didx, rows, gsem, ssem):
  c = lax.axis_index("c")
  s = lax.axis_index("s")
  pltpu.sync_copy(zeros_hbm, acc.at[pl.ds(s * RT, RT)])
  plsc.subcore_barrier()

  def pipeline(base, k):
    pltpu.sync_copy(e3.at[0, pl.ds(base, k)], sidx.at[pl.ds(0, k)])
    pltpu.sync_copy(e3.at[1, pl.ds(base, k)], didx.at[pl.ds(0, k)])
    gat, scat = [], []
    for j in range(min(PF, k)):
      gat.append(pltpu.async_copy(
          m_hbm.at[sidx.at[j]], rows.at[j % NBUF], gsem[j % NBUF]))
    for j in range(k):
      gat[j].wait()
      scat.append(pltpu.async_copy(
          rows.at[j % NBUF], acc.at[didx.at[j]], ssem, add=True))
      jn = j + PF
      if jn < k:
        if jn >= NBUF:
          scat[jn - NBUF].wait()
        gat.append(pltpu.async_copy(
            m_hbm.at[sidx.at[jn]], rows.at[jn % NBUF], gsem[jn % NBUF]))
    for j in range(max(0, k - NBUF), k):
      scat[j].wait()

  @pl.when(c == 0)
  def _():
    pipeline(s * K0, K0)

  @pl.when(c == 1)
  def _():
    pipeline(NS * K0 + s * K1, K1)

  plsc.subcore_barrier()
  pltpu.sync_copy(acc.at[pl.ds(s * RT, RT)], out.at[c, pl.ds(s * RT, RT)])


@functools.partial(
    pl.kernel,
    out_type=[_f32(NC, G, H), _f32(NC, NS, G, H)],
    mesh=_mesh,
    compiler_params=_sc_params,
    scratch_types=[
        pltpu.VMEM_SHARED((G, H), jnp.float32),
        pltpu.VMEM((G, H), jnp.float32),
        pltpu.VMEM((NODE_W, H), jnp.float32),
        pltpu.VMEM((NODE_W, H), jnp.float32),
        pltpu.VMEM((NGC, 16), jnp.int32),
        pltpu.SemaphoreType.DMA,
    ],
)
def _readout_kernel(hw_hbm, hm_hbm, gid3, zeros_hbm, neginf_hbm,
                    out_sum, out_max, acc_sum, tbl, hw_v, hm_v, gid2, ssem):
  c = lax.axis_index("c")
  s = lax.axis_index("s")
  w = c * NS + s
  pltpu.sync_copy(zeros_hbm, acc_sum.at[pl.ds(s * GS, GS)])
  pltpu.sync_copy(neginf_hbm, tbl)
  base = w * NODE_W
  pltpu.sync_copy(hw_hbm.at[pl.ds(base, NODE_W)], hw_v)
  pltpu.sync_copy(hm_hbm.at[pl.ds(base, NODE_W)], hm_v)
  pltpu.sync_copy(gid3.at[w], gid2)
  plsc.subcore_barrier()
  descrs = []
  for ch in range(NGC):
    descrs.append(pltpu.async_copy(
        hw_v.at[pl.ds(ch * 16, 16)], acc_sum.at[gid2.at[ch]], ssem, add=True))
  lanes = lax.iota(jnp.int32, 16)

  def chunk_body(ch, carry):
    gvec = gid2[ch, :]
    for j in range(16):
      gj = jnp.max(jnp.where(lanes == j, gvec, 0))
      rowidx = jnp.full((16,), gj, jnp.int32)
      node = ch * 16 + j
      for k in range(4):
        cols = lanes + (16 * k)
        cur = plsc.load_gather(tbl, [rowidx, cols])
        hv = hm_v[node, pl.ds(k * 16, 16)]
        plsc.store_scatter(tbl, [rowidx, cols], jnp.maximum(cur, hv))
    return carry

  lax.fori_loop(0, NGC, chunk_body, 0)
  for d in descrs:
    d.wait()
  pltpu.sync_copy(tbl, out_max.at[c, s])
  plsc.subcore_barrier()
  pltpu.sync_copy(acc_sum.at[pl.ds(s * GS, GS)],
                  out_sum.at[c, pl.ds(s * GS, GS)])


def _dot(a, b):
  return jnp.dot(a, b, preferred_element_type=jnp.float32,
                 precision=lax.Precision.HIGHEST)


def _dense1a_body(x_ref, w1_ref, wr1_ref, br1_ref, xw1_ref, r1_ref):
  x = x_ref[...]
  xw1_ref[...] = _dot(x, w1_ref[...])
  r1_ref[...] = jax.nn.relu(_dot(x, wr1_ref[...]) + br1_ref[...])


def _dense1b_body(dp_ref, xw1_ref, m1_ref, no_ref, ni_ref):
  dp = dp_ref[...]
  deg_out = dp[0, 0, :, 0] + dp[1, 0, :, 0]
  deg_in = dp[0, 1, :, 0] + dp[1, 1, :, 0]
  no = lax.rsqrt(jnp.maximum(deg_out, 1.0))[:, None]
  ni = lax.rsqrt(jnp.maximum(deg_in, 1.0))[:, None]
  m1_ref[...] = xw1_ref[...] * no
  no_ref[...] = jnp.broadcast_to(no, (BLK, H))
  ni_ref[...] = jnp.broadcast_to(ni, (BLK, H))


def _dense2_body(ap_ref, ni_ref, no_ref, r1_ref, b1_ref, w2_ref, wr2_ref,
                 br2_ref, m2_ref, r2_ref):
  agg = ap_ref[0] + ap_ref[1]
  h1 = jax.nn.relu(agg * ni_ref[...] + b1_ref[...]) + r1_ref[...]
  m2_ref[...] = _dot(h1, w2_ref[...]) * no_ref[...]
  r2_ref[...] = jax.nn.relu(_dot(h1, wr2_ref[...]) + br2_ref[...])


def _dense3_body(ap_ref, ni_ref, r2_ref, b2_ref, aww_ref, awb_ref,
                 h2m_ref, hw_ref):
  agg = ap_ref[0] + ap_ref[1]
  h2 = jax.nn.relu(agg * ni_ref[...] + b2_ref[...]) + r2_ref[...]
  logit = jnp.sum(h2 * aww_ref[...], axis=1, keepdims=True) + awb_ref[0, 0]
  wgt = jax.nn.sigmoid(logit)
  i = pl.program_id(0)
  rows = lax.broadcasted_iota(jnp.int32, (BLK, 1), 0) + i * BLK
  h2m_ref[...] = jnp.where(rows < N, h2, -jnp.inf)
  hw_ref[...] = jnp.where(rows < N, h2 * wgt, 0.0)


def _head_body(sp_ref, mt_ref, w3_ref, b3_ref, w4_ref, b4_ref, out_ref):
  gsum = sp_ref[0] + sp_ref[1]
  gmax = jnp.max(mt_ref[...], axis=(0, 1))
  gmax = jnp.where(jnp.isfinite(gmax), gmax, 0.0)
  hg = jnp.concatenate([gsum, gmax], axis=1)
  z = jax.nn.relu(_dot(hg, w3_ref[...]) + b3_ref[...])
  lat = _dot(z, w4_ref[...]) + b4_ref[...]
  zrow = jnp.zeros((1, DIM), jnp.float32)
  out_ref[0:10, :] = lat[0:10, :]
  out_ref[10:11, :] = zrow
  out_ref[11:100, :] = lat[10:99, :]
  out_ref[100:101, :] = zrow
  out_ref[101:300, :] = lat[99:298, :]
  out_ref[300:301, :] = zrow
  out_ref[301:NOUT, :] = lat[298:G, :]


def _full(shape):
  return pl.BlockSpec(shape, lambda i: tuple(0 for _ in shape))


_dense1a = pl.pallas_call(
    _dense1a_body,
    grid=(NP // BLK,),
    in_specs=[
        pl.BlockSpec((BLK, DIN), lambda i: (i, 0)),
        _full((DIN, H)),
        _full((DIN, H)),
        _full((1, H)),
    ],
    out_specs=[
        pl.BlockSpec((BLK, H), lambda i: (i, 0)),
        pl.BlockSpec((BLK, H), lambda i: (i, 0)),
    ],
    out_shape=[_f32(NP, H), _f32(NP, H)],
)

_dense1b = pl.pallas_call(
    _dense1b_body,
    grid=(NP // BLK,),
    in_specs=[
        pl.BlockSpec((2, 2, BLK, 16), lambda i: (0, 0, i, 0)),
        pl.BlockSpec((BLK, H), lambda i: (i, 0)),
    ],
    out_specs=[
        pl.BlockSpec((BLK, H), lambda i: (i, 0)),
        pl.BlockSpec((BLK, H), lambda i: (i, 0)),
        pl.BlockSpec((BLK, H), lambda i: (i, 0)),
    ],
    out_shape=[_f32(NP, H), _f32(NP, H), _f32(NP, H)],
)

_dense2 = pl.pallas_call(
    _dense2_body,
    grid=(NP // BLK,),
    in_specs=[
        pl.BlockSpec((2, BLK, H), lambda i: (0, i, 0)),
        pl.BlockSpec((BLK, H), lambda i: (i, 0)),
        pl.BlockSpec((BLK, H), lambda i: (i, 0)),
        pl.BlockSpec((BLK, H), lambda i: (i, 0)),
        _full((1, H)),
        _full((H, H)),
        _full((H, H)),
        _full((1, H)),
    ],
    out_specs=[
        pl.BlockSpec((BLK, H), lambda i: (i, 0)),
        pl.BlockSpec((BLK, H), lambda i: (i, 0)),
    ],
    out_shape=[_f32(NP, H), _f32(NP, H)],
)

_dense3 = pl.pallas_call(
    _dense3_body,
    grid=(NP // BLK,),
    in_specs=[
        pl.BlockSpec((2, BLK, H), lambda i: (0, i, 0)),
        pl.BlockSpec((BLK, H), lambda i: (i, 0)),
        pl.BlockSpec((BLK, H), lambda i: (i, 0)),
        _full((1, H)),
        _full((1, H)),
        _full((1, 1)),
    ],
    out_specs=[
        pl.BlockSpec((BLK, H), lambda i: (i, 0)),
        pl.BlockSpec((BLK, H), lambda i: (i, 0)),
    ],
    out_shape=[_f32(NP, H), _f32(NP, H)],
)

_head = pl.pallas_call(
    _head_body,
    grid=(1,),
    in_specs=[
        _full((NC, G, H)),
        _full((NC, NS, G, H)),
        _full((2 * H, DIM)),
        _full((1, DIM)),
        _full((DIM, DIM)),
        _full((1, DIM)),
    ],
    out_specs=_full((NOUT, DIM)),
    out_shape=_f32(NOUT, DIM),
)


def kernel(x, efeat, edge_index, graph_ids, params):
  del efeat
  f32 = jnp.float32
  xp = jnp.concatenate([x, jnp.zeros((NP - N, DIN), f32)], axis=0)
  e3 = edge_index.reshape(2, TCH, ECH)
  gid3 = jnp.concatenate(
      [graph_ids, jnp.full((NP - N,), G - 1, jnp.int32)]).reshape(NW, NGC, 16)

  ones_ech16 = jnp.ones((ECH, 16), f32)
  zeros_rt16 = jnp.zeros((RT, 16), f32)
  zeros_rth = jnp.zeros((RT, H), f32)
  zeros_gsh = jnp.zeros((GS, H), f32)
  neginf_gh = jnp.full((G, H), -jnp.inf, f32)

  p1, p2 = params['layer1'], params['layer2']

  deg_parts = _deg_kernel(e3, ones_ech16, zeros_rt16)
  xw1, r1 = _dense1a(xp, p1['W'], p1['Wr'], p1['br'].reshape(1, H))
  m1, no64, ni64 = _dense1b(deg_parts, xw1)
  agg1 = _agg_kernel(m1, e3, zeros_rth)
  m2, r2 = _dense2(agg1, ni64, no64, r1, p1['b'].reshape(1, H),
                   p2['W'], p2['Wr'], p2['br'].reshape(1, H))
  agg2 = _agg_kernel(m2, e3, zeros_rth)
  h2m, hw = _dense3(agg2, ni64, r2, p2['b'].reshape(1, H),
                    params['aw_w'].reshape(1, H),
                    params['aw_b'].reshape(1, 1))
  sum_parts, max_tbls = _readout_kernel(hw, h2m, gid3, zeros_gsh, neginf_gh)
  return _head(sum_parts, max_tbls, params['W3'],
               params['b3'].reshape(1, DIM), params['W4'],
               params['b4'].reshape(1, DIM))


# speedup vs baseline: 11.8087x; 1.0276x over previous
import functools

import jax
import jax.numpy as jnp
from jax import lax
from jax.experimental import pallas as pl
from jax.experimental.pallas import tpu as pltpu
from jax.experimental.pallas import tpu_sc as plsc

N = 10000
NP = 10240
E = 320000
DIN = 128
H = 64
DIM = 128
G = 512
NC = 2
NS = 16
NW = NC * NS
ECH = 125
TCH = E // ECH
NCHUNK = TCH // NW
K0 = 129
K1 = 31
RT = NP // NS
NODE_W = NP // NW
NGC = NODE_W // 16
GS = G // NS
BLK = 1024
NBUF = 6
PF = 4
NOUT = 515

_mesh = plsc.VectorSubcoreMesh(
    core_axis_name="c", subcore_axis_name="s", num_cores=NC, num_subcores=NS)
_sc_params = pltpu.CompilerParams(
    use_tc_tiling_on_sc=False, needs_layout_passes=False)


def _f32(*shape):
  return jax.ShapeDtypeStruct(shape, jnp.float32)


@functools.partial(
    pl.kernel,
    out_type=_f32(NC, 2, NP // 128, 128),
    mesh=_mesh,
    compiler_params=_sc_params,
    scratch_types=[
        pltpu.VMEM_SHARED((NP,), jnp.float32),
        pltpu.VMEM_SHARED((NP,), jnp.float32),
        pltpu.VMEM((NCHUNK, ECH), jnp.int32),
        pltpu.VMEM((NCHUNK, ECH), jnp.int32),
        pltpu.VMEM((ECH,), jnp.float32),
        pltpu.VMEM((RT,), jnp.float32),
        pltpu.VMEM((RT // 128, 128), jnp.float32),
        pltpu.SemaphoreType.DMA,
    ],
)
def _deg_kernel(e3, ones_hbm, zeros_hbm, out,
                acc_out, acc_in, sidx, didx, ones_v, tmp1, tmp2, ssem):
  c = lax.axis_index("c")
  s = lax.axis_index("s")
  w = c * NS + s
  pltpu.sync_copy(zeros_hbm, acc_out.at[pl.ds(s * RT, RT)])
  pltpu.sync_copy(zeros_hbm, acc_in.at[pl.ds(s * RT, RT)])
  pltpu.sync_copy(ones_hbm, ones_v)
  pltpu.sync_copy(e3.at[0, pl.ds(w * NCHUNK, NCHUNK)], sidx)
  pltpu.sync_copy(e3.at[1, pl.ds(w * NCHUNK, NCHUNK)], didx)
  plsc.subcore_barrier()
  descrs = []
  for j in range(NCHUNK):
    descrs.append(
        pltpu.async_copy(ones_v, acc_out.at[sidx.at[j]], ssem, add=True))
    descrs.append(
        pltpu.async_copy(ones_v, acc_in.at[didx.at[j]], ssem, add=True))
  for d in descrs:
    d.wait()
  plsc.subcore_barrier()
  for di, acc in enumerate((acc_out, acc_in)):
    pltpu.sync_copy(acc.at[pl.ds(s * RT, RT)], tmp1)
    for k in range(RT // 16):
      tmp2[k // 8, pl.ds((k % 8) * 16, 16)] = tmp1[pl.ds(k * 16, 16)]
    pltpu.sync_copy(tmp2, out.at[c, di, pl.ds(s * (RT // 128), RT // 128)])


@functools.partial(
    pl.kernel,
    out_type=_f32(NC, NP, H),
    mesh=_mesh,
    compiler_params=_sc_params,
    scratch_types=[
        pltpu.VMEM_SHARED((NP, H), jnp.float32),
        pltpu.VMEM((K0, ECH), jnp.int32),
        pltpu.VMEM((K0, ECH), jnp.int32),
        pltpu.VMEM((NBUF, ECH, H), jnp.float32),
        [pltpu.SemaphoreType.DMA] * NBUF,
        pltpu.SemaphoreType.DMA,
    ],
)
def _agg_kernel(m_hbm, e3, zeros_hbm, out,
                acc, sidx, didx, rows, gsem, ssem):
  c = lax.axis_index("c")
  s = lax.axis_index("s")
  pltpu.sync_copy(zeros_hbm, acc.at[pl.ds(s * RT, RT)])
  plsc.subcore_barrier()

  def pipeline(base, k):
    pltpu.sync_copy(e3.at[0, pl.ds(base, k)], sidx.at[pl.ds(0, k)])
    pltpu.sync_copy(e3.at[1, pl.ds(base, k)], didx.at[pl.ds(0, k)])
    gat, scat = [], []
    for j in range(min(PF, k)):
      gat.append(pltpu.async_copy(
          m_hbm.at[sidx.at[j]], rows.at[j % NBUF], gsem[j % NBUF]))
    for j in range(k):
      gat[j].wait()
      scat.append(pltpu.async_copy(
          rows.at[j % NBUF], acc.at[didx.at[j]], ssem, add=True))
      jn = j + PF
      if jn < k:
        if jn >= NBUF:
          scat[jn - NBUF].wait()
        gat.append(pltpu.async_copy(
            m_hbm.at[sidx.at[jn]], rows.at[jn % NBUF], gsem[jn % NBUF]))
    for j in range(max(0, k - NBUF), k):
      scat[j].wait()

  @pl.when(c == 0)
  def _():
    pipeline(s * K0, K0)

  @pl.when(c == 1)
  def _():
    pipeline(NS * K0 + s * K1, K1)

  plsc.subcore_barrier()
  pltpu.sync_copy(acc.at[pl.ds(s * RT, RT)], out.at[c, pl.ds(s * RT, RT)])


@functools.partial(
    pl.kernel,
    out_type=[_f32(NC, G, H), _f32(NC, NS, G, H)],
    mesh=_mesh,
    compiler_params=_sc_params,
    scratch_types=[
        pltpu.VMEM_SHARED((G, H), jnp.float32),
        pltpu.VMEM((G, H), jnp.float32),
        pltpu.VMEM((NODE_W, H), jnp.float32),
        pltpu.VMEM((NODE_W, H), jnp.float32),
        pltpu.VMEM((NGC, 16), jnp.int32),
        pltpu.SemaphoreType.DMA,
    ],
)
def _readout_kernel(hw_hbm, hm_hbm, gid3, zeros_hbm, neginf_hbm,
                    out_sum, out_max, acc_sum, tbl, hw_v, hm_v, gid2, ssem):
  c = lax.axis_index("c")
  s = lax.axis_index("s")
  w = c * NS + s
  pltpu.sync_copy(zeros_hbm, acc_sum.at[pl.ds(s * GS, GS)])
  pltpu.sync_copy(neginf_hbm, tbl)
  base = w * NODE_W
  pltpu.sync_copy(hw_hbm.at[pl.ds(base, NODE_W)], hw_v)
  pltpu.sync_copy(hm_hbm.at[pl.ds(base, NODE_W)], hm_v)
  pltpu.sync_copy(gid3.at[w], gid2)
  plsc.subcore_barrier()
  descrs = []
  for ch in range(NGC):
    descrs.append(pltpu.async_copy(
        hw_v.at[pl.ds(ch * 16, 16)], acc_sum.at[gid2.at[ch]], ssem, add=True))
  lanes = lax.iota(jnp.int32, 16)

  def chunk_body(ch, carry):
    gvec = gid2[ch, :]
    for j in range(16):
      gj = jnp.max(jnp.where(lanes == j, gvec, 0))
      rowidx = jnp.full((16,), gj, jnp.int32)
      node = ch * 16 + j
      for k in range(4):
        cols = lanes + (16 * k)
        cur = plsc.load_gather(tbl, [rowidx, cols])
        hv = hm_v[node, pl.ds(k * 16, 16)]
        plsc.store_scatter(tbl, [rowidx, cols], jnp.maximum(cur, hv))
    return carry

  lax.fori_loop(0, NGC, chunk_body, 0)
  for d in descrs:
    d.wait()
  pltpu.sync_copy(tbl, out_max.at[c, s])
  plsc.subcore_barrier()
  pltpu.sync_copy(acc_sum.at[pl.ds(s * GS, GS)],
                  out_sum.at[c, pl.ds(s * GS, GS)])


def _dot(a, b):
  return jnp.dot(a, b, preferred_element_type=jnp.float32,
                 precision=lax.Precision.HIGHEST)


def _dense1a_body(x_ref, w1_ref, wr1_ref, br1_ref, xw1_ref, r1_ref):
  x = x_ref[...]
  xw1_ref[...] = _dot(x, w1_ref[...])
  r1_ref[...] = jax.nn.relu(_dot(x, wr1_ref[...]) + br1_ref[...])


def _unpack_col(v):
  v3 = jnp.broadcast_to(v[:, None, :], (8, 128, 128)).reshape(BLK, 128)
  lane = lax.broadcasted_iota(jnp.int32, (BLK, 128), 1)
  row = lax.broadcasted_iota(jnp.int32, (BLK, 128), 0)
  sel = lane == (row % 128)
  return jnp.sum(jnp.where(sel, v3, 0.0), axis=1, keepdims=True)


def _dense1b_body(dp_ref, xw1_ref, m1_ref, no_ref, ni_ref):
  dp = dp_ref[...]
  no = lax.rsqrt(jnp.maximum(_unpack_col(dp[0, 0] + dp[1, 0]), 1.0))
  ni = lax.rsqrt(jnp.maximum(_unpack_col(dp[0, 1] + dp[1, 1]), 1.0))
  m1_ref[...] = xw1_ref[...] * no
  no_ref[...] = jnp.broadcast_to(no, (BLK, H))
  ni_ref[...] = jnp.broadcast_to(ni, (BLK, H))


def _dense2_body(ap_ref, ni_ref, no_ref, r1_ref, b1_ref, w2_ref, wr2_ref,
                 br2_ref, m2_ref, r2_ref):
  agg = ap_ref[0] + ap_ref[1]
  h1 = jax.nn.relu(agg * ni_ref[...] + b1_ref[...]) + r1_ref[...]
  m2_ref[...] = _dot(h1, w2_ref[...]) * no_ref[...]
  r2_ref[...] = jax.nn.relu(_dot(h1, wr2_ref[...]) + br2_ref[...])


def _dense3_body(ap_ref, ni_ref, r2_ref, b2_ref, aww_ref, awb_ref,
                 h2m_ref, hw_ref):
  agg = ap_ref[0] + ap_ref[1]
  h2 = jax.nn.relu(agg * ni_ref[...] + b2_ref[...]) + r2_ref[...]
  logit = jnp.sum(h2 * aww_ref[...], axis=1, keepdims=True) + awb_ref[0, 0]
  wgt = jax.nn.sigmoid(logit)
  i = pl.program_id(0)
  rows = lax.broadcasted_iota(jnp.int32, (BLK, 1), 0) + i * BLK
  h2m_ref[...] = jnp.where(rows < N, h2, -jnp.inf)
  hw_ref[...] = jnp.where(rows < N, h2 * wgt, 0.0)


def _head_body(sp_ref, mt_ref, w3_ref, b3_ref, w4_ref, b4_ref, out_ref):
  gsum = sp_ref[0] + sp_ref[1]
  gmax = jnp.max(mt_ref[...], axis=(0, 1))
  gmax = jnp.where(jnp.isfinite(gmax), gmax, 0.0)
  hg = jnp.concatenate([gsum, gmax], axis=1)
  z = jax.nn.relu(_dot(hg, w3_ref[...]) + b3_ref[...])
  lat = _dot(z, w4_ref[...]) + b4_ref[...]
  zrow = jnp.zeros((1, DIM), jnp.float32)
  out_ref[0:10, :] = lat[0:10, :]
  out_ref[10:11, :] = zrow
  out_ref[11:100, :] = lat[10:99, :]
  out_ref[100:101, :] = zrow
  out_ref[101:300, :] = lat[99:298, :]
  out_ref[300:301, :] = zrow
  out_ref[301:NOUT, :] = lat[298:G, :]


def _full(shape):
  return pl.BlockSpec(shape, lambda i: tuple(0 for _ in shape))


_dense1a = pl.pallas_call(
    _dense1a_body,
    grid=(NP // BLK,),
    in_specs=[
        pl.BlockSpec((BLK, DIN), lambda i: (i, 0)),
        _full((DIN, H)),
        _full((DIN, H)),
        _full((1, H)),
    ],
    out_specs=[
        pl.BlockSpec((BLK, H), lambda i: (i, 0)),
        pl.BlockSpec((BLK, H), lambda i: (i, 0)),
    ],
    out_shape=[_f32(NP, H), _f32(NP, H)],
)

_dense1b = pl.pallas_call(
    _dense1b_body,
    grid=(NP // BLK,),
    in_specs=[
        pl.BlockSpec((2, 2, BLK // 128, 128), lambda i: (0, 0, i, 0)),
        pl.BlockSpec((BLK, H), lambda i: (i, 0)),
    ],
    out_specs=[
        pl.BlockSpec((BLK, H), lambda i: (i, 0)),
        pl.BlockSpec((BLK, H), lambda i: (i, 0)),
        pl.BlockSpec((BLK, H), lambda i: (i, 0)),
    ],
    out_shape=[_f32(NP, H), _f32(NP, H), _f32(NP, H)],
)

_dense2 = pl.pallas_call(
    _dense2_body,
    grid=(NP // BLK,),
    in_specs=[
        pl.BlockSpec((2, BLK, H), lambda i: (0, i, 0)),
        pl.BlockSpec((BLK, H), lambda i: (i, 0)),
        pl.BlockSpec((BLK, H), lambda i: (i, 0)),
        pl.BlockSpec((BLK, H), lambda i: (i, 0)),
        _full((1, H)),
        _full((H, H)),
        _full((H, H)),
        _full((1, H)),
    ],
    out_specs=[
        pl.BlockSpec((BLK, H), lambda i: (i, 0)),
        pl.BlockSpec((BLK, H), lambda i: (i, 0)),
    ],
    out_shape=[_f32(NP, H), _f32(NP, H)],
)

_dense3 = pl.pallas_call(
    _dense3_body,
    grid=(NP // BLK,),
    in_specs=[
        pl.BlockSpec((2, BLK, H), lambda i: (0, i, 0)),
        pl.BlockSpec((BLK, H), lambda i: (i, 0)),
        pl.BlockSpec((BLK, H), lambda i: (i, 0)),
        _full((1, H)),
        _full((1, H)),
        _full((1, 1)),
    ],
    out_specs=[
        pl.BlockSpec((BLK, H), lambda i: (i, 0)),
        pl.BlockSpec((BLK, H), lambda i: (i, 0)),
    ],
    out_shape=[_f32(NP, H), _f32(NP, H)],
)

_head = pl.pallas_call(
    _head_body,
    grid=(1,),
    in_specs=[
        _full((NC, G, H)),
        _full((NC, NS, G, H)),
        _full((2 * H, DIM)),
        _full((1, DIM)),
        _full((DIM, DIM)),
        _full((1, DIM)),
    ],
    out_specs=_full((NOUT, DIM)),
    out_shape=_f32(NOUT, DIM),
)


def kernel(x, efeat, edge_index, graph_ids, params):
  del efeat
  f32 = jnp.float32
  xp = jnp.concatenate([x, jnp.zeros((NP - N, DIN), f32)], axis=0)
  e3 = edge_index.reshape(2, TCH, ECH)
  gid3 = jnp.concatenate(
      [graph_ids, jnp.full((NP - N,), G - 1, jnp.int32)]).reshape(NW, NGC, 16)

  ones_ech = jnp.ones((ECH,), f32)
  zeros_rt = jnp.zeros((RT,), f32)
  zeros_rth = jnp.zeros((RT, H), f32)
  zeros_gsh = jnp.zeros((GS, H), f32)
  neginf_gh = jnp.full((G, H), -jnp.inf, f32)

  p1, p2 = params['layer1'], params['layer2']

  deg_parts = _deg_kernel(e3, ones_ech, zeros_rt)
  xw1, r1 = _dense1a(xp, p1['W'], p1['Wr'], p1['br'].reshape(1, H))
  m1, no64, ni64 = _dense1b(deg_parts, xw1)
  agg1 = _agg_kernel(m1, e3, zeros_rth)
  m2, r2 = _dense2(agg1, ni64, no64, r1, p1['b'].reshape(1, H),
                   p2['W'], p2['Wr'], p2['br'].reshape(1, H))
  agg2 = _agg_kernel(m2, e3, zeros_rth)
  h2m, hw = _dense3(agg2, ni64, r2, p2['b'].reshape(1, H),
                    params['aw_w'].reshape(1, H),
                    params['aw_b'].reshape(1, 1))
  sum_parts, max_tbls = _readout_kernel(hw, h2m, gid3, zeros_gsh, neginf_gh)
  return _head(sum_parts, max_tbls, params['W3'],
               params['b3'].reshape(1, DIM), params['W4'],
               params['b4'].reshape(1, DIM))


# speedup vs baseline: 13.9911x; 1.1848x over previous
import functools

import jax
import jax.numpy as jnp
from jax import lax
from jax.experimental import pallas as pl
from jax.experimental.pallas import tpu as pltpu
from jax.experimental.pallas import tpu_sc as plsc

N = 10000
NP = 10240
E = 320000
DIN = 128
H = 64
DIM = 128
G = 512
NC = 2
NS = 16
NW = NC * NS
ECH = 125
TCH = E // ECH
NCHUNK = TCH // NW
K0 = 80
K1 = 80
RT = NP // NS
NODE_W = NP // NW
NGC = NODE_W // 16
GS = G // NS
BLK = 1024
NBUF = 7
PF = 5
NOUT = 515

_mesh = plsc.VectorSubcoreMesh(
    core_axis_name="c", subcore_axis_name="s", num_cores=NC, num_subcores=NS)
_sc_params = pltpu.CompilerParams(
    use_tc_tiling_on_sc=False, needs_layout_passes=False)


def _f32(*shape):
  return jax.ShapeDtypeStruct(shape, jnp.float32)


@functools.partial(
    pl.kernel,
    out_type=_f32(NC, 2, NP // 128, 128),
    mesh=_mesh,
    compiler_params=_sc_params,
    scratch_types=[
        pltpu.VMEM_SHARED((NP,), jnp.float32),
        pltpu.VMEM_SHARED((NP,), jnp.float32),
        pltpu.VMEM((NCHUNK, ECH), jnp.int32),
        pltpu.VMEM((NCHUNK, ECH), jnp.int32),
        pltpu.VMEM((ECH,), jnp.float32),
        pltpu.VMEM((RT,), jnp.float32),
        pltpu.VMEM((RT // 128, 128), jnp.float32),
        pltpu.SemaphoreType.DMA,
    ],
)
def _deg_kernel(e3, ones_hbm, zeros_hbm, out,
                acc_out, acc_in, sidx, didx, ones_v, tmp1, tmp2, ssem):
  c = lax.axis_index("c")
  s = lax.axis_index("s")
  w = c * NS + s
  pltpu.sync_copy(zeros_hbm, acc_out.at[pl.ds(s * RT, RT)])
  pltpu.sync_copy(zeros_hbm, acc_in.at[pl.ds(s * RT, RT)])
  pltpu.sync_copy(ones_hbm, ones_v)
  pltpu.sync_copy(e3.at[0, pl.ds(w * NCHUNK, NCHUNK)], sidx)
  pltpu.sync_copy(e3.at[1, pl.ds(w * NCHUNK, NCHUNK)], didx)
  plsc.subcore_barrier()
  descrs = []
  for j in range(NCHUNK):
    descrs.append(
        pltpu.async_copy(ones_v, acc_out.at[sidx.at[j]], ssem, add=True))
    descrs.append(
        pltpu.async_copy(ones_v, acc_in.at[didx.at[j]], ssem, add=True))
  for d in descrs:
    d.wait()
  plsc.subcore_barrier()
  for di, acc in enumerate((acc_out, acc_in)):
    pltpu.sync_copy(acc.at[pl.ds(s * RT, RT)], tmp1)
    for k in range(RT // 16):
      tmp2[k // 8, pl.ds((k % 8) * 16, 16)] = tmp1[pl.ds(k * 16, 16)]
    pltpu.sync_copy(tmp2, out.at[c, di, pl.ds(s * (RT // 128), RT // 128)])


@functools.partial(
    pl.kernel,
    out_type=_f32(NC, NP, H),
    mesh=_mesh,
    compiler_params=_sc_params,
    scratch_types=[
        pltpu.VMEM_SHARED((NP, H), jnp.float32),
        pltpu.VMEM((K0, ECH), jnp.int32),
        pltpu.VMEM((K0, ECH), jnp.int32),
        pltpu.VMEM((NBUF, ECH, H), jnp.float32),
        [pltpu.SemaphoreType.DMA] * NBUF,
        pltpu.SemaphoreType.DMA,
    ],
)
def _agg_kernel(m_hbm, e3, zeros_hbm, out,
                acc, sidx, didx, rows, gsem, ssem):
  c = lax.axis_index("c")
  s = lax.axis_index("s")
  pltpu.sync_copy(zeros_hbm, acc.at[pl.ds(s * RT, RT)])
  plsc.subcore_barrier()

  def pipeline(base, k):
    pltpu.sync_copy(e3.at[0, pl.ds(base, k)], sidx.at[pl.ds(0, k)])
    pltpu.sync_copy(e3.at[1, pl.ds(base, k)], didx.at[pl.ds(0, k)])
    gat, scat = [], []
    for j in range(min(PF, k)):
      gat.append(pltpu.async_copy(
          m_hbm.at[sidx.at[j]], rows.at[j % NBUF], gsem[j % NBUF]))
    for j in range(k):
      gat[j].wait()
      scat.append(pltpu.async_copy(
          rows.at[j % NBUF], acc.at[didx.at[j]], ssem, add=True))
      jn = j + PF
      if jn < k:
        if jn >= NBUF:
          scat[jn - NBUF].wait()
        gat.append(pltpu.async_copy(
            m_hbm.at[sidx.at[jn]], rows.at[jn % NBUF], gsem[jn % NBUF]))
    for j in range(max(0, k - NBUF), k):
      scat[j].wait()

  @pl.when(c == 0)
  def _():
    pipeline(s * K0, K0)

  @pl.when(c == 1)
  def _():
    pipeline(NS * K0 + s * K1, K1)

  plsc.subcore_barrier()
  pltpu.sync_copy(acc.at[pl.ds(s * RT, RT)], out.at[c, pl.ds(s * RT, RT)])


@functools.partial(
    pl.kernel,
    out_type=[_f32(NC, G, H), _f32(NC, NS, G, H)],
    mesh=_mesh,
    compiler_params=_sc_params,
    scratch_types=[
        pltpu.VMEM_SHARED((G, H), jnp.float32),
        pltpu.VMEM((G, H), jnp.float32),
        pltpu.VMEM((NODE_W, H), jnp.float32),
        pltpu.VMEM((NODE_W, H), jnp.float32),
        pltpu.VMEM((NGC, 16), jnp.int32),
        pltpu.SemaphoreType.DMA,
    ],
)
def _readout_kernel(hw_hbm, hm_hbm, gid3, zeros_hbm, neginf_hbm,
                    out_sum, out_max, acc_sum, tbl, hw_v, hm_v, gid2, ssem):
  c = lax.axis_index("c")
  s = lax.axis_index("s")
  w = c * NS + s
  pltpu.sync_copy(zeros_hbm, acc_sum.at[pl.ds(s * GS, GS)])
  pltpu.sync_copy(neginf_hbm, tbl)
  base = w * NODE_W
  pltpu.sync_copy(hw_hbm.at[pl.ds(base, NODE_W)], hw_v)
  pltpu.sync_copy(hm_hbm.at[pl.ds(base, NODE_W)], hm_v)
  pltpu.sync_copy(gid3.at[w], gid2)
  plsc.subcore_barrier()
  descrs = []
  for ch in range(NGC):
    descrs.append(pltpu.async_copy(
        hw_v.at[pl.ds(ch * 16, 16)], acc_sum.at[gid2.at[ch]], ssem, add=True))
  lanes = lax.iota(jnp.int32, 16)

  def chunk_body(ch, carry):
    gvec = gid2[ch, :]
    for j in range(16):
      gj = jnp.max(jnp.where(lanes == j, gvec, 0))
      rowidx = jnp.full((16,), gj, jnp.int32)
      node = ch * 16 + j
      for k in range(4):
        cols = lanes + (16 * k)
        cur = plsc.load_gather(tbl, [rowidx, cols])
        hv = hm_v[node, pl.ds(k * 16, 16)]
        plsc.store_scatter(tbl, [rowidx, cols], jnp.maximum(cur, hv))
    return carry

  lax.fori_loop(0, NGC, chunk_body, 0)
  for d in descrs:
    d.wait()
  pltpu.sync_copy(tbl, out_max.at[c, s])
  plsc.subcore_barrier()
  pltpu.sync_copy(acc_sum.at[pl.ds(s * GS, GS)],
                  out_sum.at[c, pl.ds(s * GS, GS)])


def _dot(a, b):
  return jnp.dot(a, b, preferred_element_type=jnp.float32,
                 precision=lax.Precision.HIGHEST)


def _dense1a_body(x_ref, w1_ref, wr1_ref, br1_ref, xw1_ref, r1_ref):
  x = x_ref[...]
  xw1_ref[...] = _dot(x, w1_ref[...])
  r1_ref[...] = jax.nn.relu(_dot(x, wr1_ref[...]) + br1_ref[...])


def _unpack_col(v):
  v3 = jnp.broadcast_to(v[:, None, :], (8, 128, 128)).reshape(BLK, 128)
  lane = lax.broadcasted_iota(jnp.int32, (BLK, 128), 1)
  row = lax.broadcasted_iota(jnp.int32, (BLK, 128), 0)
  sel = lane == (row % 128)
  return jnp.sum(jnp.where(sel, v3, 0.0), axis=1, keepdims=True)


def _dense1b_body(dp_ref, xw1_ref, m1_ref, no_ref, ni_ref):
  dp = dp_ref[...]
  no = lax.rsqrt(jnp.maximum(_unpack_col(dp[0, 0] + dp[1, 0]), 1.0))
  ni = lax.rsqrt(jnp.maximum(_unpack_col(dp[0, 1] + dp[1, 1]), 1.0))
  m1_ref[...] = xw1_ref[...] * no
  no_ref[...] = jnp.broadcast_to(no, (BLK, H))
  ni_ref[...] = jnp.broadcast_to(ni, (BLK, H))


def _dense2_body(ap_ref, ni_ref, no_ref, r1_ref, b1_ref, w2_ref, wr2_ref,
                 br2_ref, m2_ref, r2_ref):
  agg = ap_ref[0] + ap_ref[1]
  h1 = jax.nn.relu(agg * ni_ref[...] + b1_ref[...]) + r1_ref[...]
  m2_ref[...] = _dot(h1, w2_ref[...]) * no_ref[...]
  r2_ref[...] = jax.nn.relu(_dot(h1, wr2_ref[...]) + br2_ref[...])


def _dense3_body(ap_ref, ni_ref, r2_ref, b2_ref, aww_ref, awb_ref,
                 h2m_ref, hw_ref):
  agg = ap_ref[0] + ap_ref[1]
  h2 = jax.nn.relu(agg * ni_ref[...] + b2_ref[...]) + r2_ref[...]
  logit = jnp.sum(h2 * aww_ref[...], axis=1, keepdims=True) + awb_ref[0, 0]
  wgt = jax.nn.sigmoid(logit)
  i = pl.program_id(0)
  rows = lax.broadcasted_iota(jnp.int32, (BLK, 1), 0) + i * BLK
  h2m_ref[...] = jnp.where(rows < N, h2, -jnp.inf)
  hw_ref[...] = jnp.where(rows < N, h2 * wgt, 0.0)


def _head_body(sp_ref, mt_ref, w3_ref, b3_ref, w4_ref, b4_ref, out_ref):
  gsum = sp_ref[0] + sp_ref[1]
  gmax = jnp.max(mt_ref[...], axis=(0, 1))
  gmax = jnp.where(jnp.isfinite(gmax), gmax, 0.0)
  hg = jnp.concatenate([gsum, gmax], axis=1)
  z = jax.nn.relu(_dot(hg, w3_ref[...]) + b3_ref[...])
  lat = _dot(z, w4_ref[...]) + b4_ref[...]
  zrow = jnp.zeros((1, DIM), jnp.float32)
  out_ref[0:10, :] = lat[0:10, :]
  out_ref[10:11, :] = zrow
  out_ref[11:100, :] = lat[10:99, :]
  out_ref[100:101, :] = zrow
  out_ref[101:300, :] = lat[99:298, :]
  out_ref[300:301, :] = zrow
  out_ref[301:NOUT, :] = lat[298:G, :]


def _full(shape):
  return pl.BlockSpec(shape, lambda i: tuple(0 for _ in shape))


_dense1a = pl.pallas_call(
    _dense1a_body,
    grid=(NP // BLK,),
    in_specs=[
        pl.BlockSpec((BLK, DIN), lambda i: (i, 0)),
        _full((DIN, H)),
        _full((DIN, H)),
        _full((1, H)),
    ],
    out_specs=[
        pl.BlockSpec((BLK, H), lambda i: (i, 0)),
        pl.BlockSpec((BLK, H), lambda i: (i, 0)),
    ],
    out_shape=[_f32(NP, H), _f32(NP, H)],
)

_dense1b = pl.pallas_call(
    _dense1b_body,
    grid=(NP // BLK,),
    in_specs=[
        pl.BlockSpec((2, 2, BLK // 128, 128), lambda i: (0, 0, i, 0)),
        pl.BlockSpec((BLK, H), lambda i: (i, 0)),
    ],
    out_specs=[
        pl.BlockSpec((BLK, H), lambda i: (i, 0)),
        pl.BlockSpec((BLK, H), lambda i: (i, 0)),
        pl.BlockSpec((BLK, H), lambda i: (i, 0)),
    ],
    out_shape=[_f32(NP, H), _f32(NP, H), _f32(NP, H)],
)

_dense2 = pl.pallas_call(
    _dense2_body,
    grid=(NP // BLK,),
    in_specs=[
        pl.BlockSpec((2, BLK, H), lambda i: (0, i, 0)),
        pl.BlockSpec((BLK, H), lambda i: (i, 0)),
        pl.BlockSpec((BLK, H), lambda i: (i, 0)),
        pl.BlockSpec((BLK, H), lambda i: (i, 0)),
        _full((1, H)),
        _full((H, H)),
        _full((H, H)),
        _full((1, H)),
    ],
    out_specs=[
        pl.BlockSpec((BLK, H), lambda i: (i, 0)),
        pl.BlockSpec((BLK, H), lambda i: (i, 0)),
    ],
    out_shape=[_f32(NP, H), _f32(NP, H)],
)

_dense3 = pl.pallas_call(
    _dense3_body,
    grid=(NP // BLK,),
    in_specs=[
        pl.BlockSpec((2, BLK, H), lambda i: (0, i, 0)),
        pl.BlockSpec((BLK, H), lambda i: (i, 0)),
        pl.BlockSpec((BLK, H), lambda i: (i, 0)),
        _full((1, H)),
        _full((1, H)),
        _full((1, 1)),
    ],
    out_specs=[
        pl.BlockSpec((BLK, H), lambda i: (i, 0)),
        pl.BlockSpec((BLK, H), lambda i: (i, 0)),
    ],
    out_shape=[_f32(NP, H), _f32(NP, H)],
)

_head = pl.pallas_call(
    _head_body,
    grid=(1,),
    in_specs=[
        _full((NC, G, H)),
        _full((NC, NS, G, H)),
        _full((2 * H, DIM)),
        _full((1, DIM)),
        _full((DIM, DIM)),
        _full((1, DIM)),
    ],
    out_specs=_full((NOUT, DIM)),
    out_shape=_f32(NOUT, DIM),
)


def kernel(x, efeat, edge_index, graph_ids, params):
  del efeat
  f32 = jnp.float32
  xp = jnp.concatenate([x, jnp.zeros((NP - N, DIN), f32)], axis=0)
  e3 = edge_index.reshape(2, TCH, ECH)
  gid3 = jnp.concatenate(
      [graph_ids, jnp.full((NP - N,), G - 1, jnp.int32)]).reshape(NW, NGC, 16)

  ones_ech = jnp.ones((ECH,), f32)
  zeros_rt = jnp.zeros((RT,), f32)
  zeros_rth = jnp.zeros((RT, H), f32)
  zeros_gsh = jnp.zeros((GS, H), f32)
  neginf_gh = jnp.full((G, H), -jnp.inf, f32)

  p1, p2 = params['layer1'], params['layer2']

  deg_parts = _deg_kernel(e3, ones_ech, zeros_rt)
  xw1, r1 = _dense1a(xp, p1['W'], p1['Wr'], p1['br'].reshape(1, H))
  m1, no64, ni64 = _dense1b(deg_parts, xw1)
  agg1 = _agg_kernel(m1, e3, zeros_rth)
  m2, r2 = _dense2(agg1, ni64, no64, r1, p1['b'].reshape(1, H),
                   p2['W'], p2['Wr'], p2['br'].reshape(1, H))
  agg2 = _agg_kernel(m2, e3, zeros_rth)
  h2m, hw = _dense3(agg2, ni64, r2, p2['b'].reshape(1, H),
                    params['aw_w'].reshape(1, H),
                    params['aw_b'].reshape(1, 1))
  sum_parts, max_tbls = _readout_kernel(hw, h2m, gid3, zeros_gsh, neginf_gh)
  return _head(sum_parts, max_tbls, params['W3'],
               params['b3'].reshape(1, DIM), params['W4'],
               params['b4'].reshape(1, DIM))
